# baseline (speedup 1.0000x reference)
import jax
import jax.numpy as jnp
from jax import lax
from jax.experimental import pallas as pl

N = 4096
HID = 256
TOP_K = 10
BLK_A = 512
BLK_B = 256
BLK_C = 256
SENT = -1e9


def _dense_body(x0, x1, x2, w01, w02, w11, w12, w21, w22, aw, fw, fb,
                nt_out, rsum_out):
    a = aw[...]
    a = a - jnp.max(a, axis=1, keepdims=True)
    e = jnp.exp(a)
    a = e / jnp.sum(e, axis=1, keepdims=True)

    def mlp(x, w1, w2):
        h = jnp.maximum(jnp.dot(x[...], w1[...], preferred_element_type=jnp.float32), 0.0)
        return jnp.dot(h, w2[...], preferred_element_type=jnp.float32)

    f0 = mlp(x0, w01, w02)
    f1 = mlp(x1, w11, w12)
    f2 = mlp(x2, w21, w22)
    fsum = a[0, 0] * f0 + a[0, 1] * f1 + a[0, 2] * f2
    fused = jnp.dot(fsum, fw[...], preferred_element_type=jnp.float32) + fb[...]
    nrm = jnp.sqrt(jnp.sum(fused * fused, axis=1, keepdims=True))
    nrm = jnp.maximum(nrm, 1e-12)
    nt_out[...] = (fused / nrm).T
    rsum_out[...] = jnp.sum(fused, axis=1, keepdims=True)


def _topk_body(nb, nt, rsum, et_out, vals_out, w_out):
    i = pl.program_id(0)
    r0 = i * BLK_B
    rids_i = r0 + lax.broadcasted_iota(jnp.int32, (BLK_B, 1), 0)
    rids = rids_i.astype(jnp.float32)
    cols = lax.broadcasted_iota(jnp.int32, (BLK_B, N), 1).astype(jnp.float32)
    sim = lax.dot_general(nb[...], nt[...], (((0,), (0,)), ((), ())),
                          preferred_element_type=jnp.float32)
    sim = jnp.where(cols == rids, SENT, sim)
    picks = [rids]
    for _ in range(TOP_K):
        m = jnp.max(sim, axis=1, keepdims=True)
        cand = jnp.where(sim == m, cols, float(N))
        j = jnp.min(cand, axis=1, keepdims=True)
        sim = jnp.where(cols == j, SENT, sim)
        picks.append(j)
    selmask = (sim == SENT).astype(jnp.float32)
    msum = jnp.dot(selmask, rsum[...], preferred_element_type=jnp.float32)
    w = jax.nn.sigmoid(msum / float((TOP_K + 1) * HID))
    keep = w > 0.0
    picks += [jnp.full((BLK_B, 1), -1.0, jnp.float32)] * (16 - len(picks))
    edges = jnp.concatenate(picks, axis=1).astype(jnp.int32)
    et_out[...] = edges.T
    vals_out[...] = jnp.where(keep, 1.0, 0.0)
    w_out[...] = jnp.where(keep, w, 0.0)


def _hbuild_body(et, keepf, h_out):
    i = pl.program_id(0)
    r0 = i * BLK_C
    rids = r0 + lax.broadcasted_iota(jnp.int32, (BLK_C, 1), 0)
    acc = et[0:1, :] == rids
    for j in range(1, TOP_K + 1):
        acc = jnp.logical_or(acc, et[j:j + 1, :] == rids)
    h_out[...] = jnp.where(acc, keepf[...], 0.0)


def kernel(x0, x1, x2, mW0_1, mb0_1, mW0_2, mb0_2, mW1_1, mb1_1, mW1_2, mb1_2,
           mW2_1, mb2_1, mW2_2, mb2_2, attn_weights, fW, fb):
    f32 = jnp.float32
    aw2 = attn_weights.reshape(1, 3)
    fb2 = fb.reshape(1, HID)

    whole = lambda shape: pl.BlockSpec(shape, lambda i: (0, 0))
    rows = lambda w: pl.BlockSpec((BLK_A, w), lambda i: (i, 0))

    normed_t, rsum = pl.pallas_call(
        _dense_body,
        grid=(N // BLK_A,),
        in_specs=[
            rows(256), rows(512), rows(128),
            whole((256, HID)), whole((HID, HID)),
            whole((512, HID)), whole((HID, HID)),
            whole((128, HID)), whole((HID, HID)),
            whole((1, 3)), whole((HID, HID)), whole((1, HID)),
        ],
        out_specs=[pl.BlockSpec((HID, BLK_A), lambda i: (0, i)), rows(1)],
        out_shape=[
            jax.ShapeDtypeStruct((HID, N), f32),
            jax.ShapeDtypeStruct((N, 1), f32),
        ],
    )(x0, x1, x2, mW0_1, mW0_2, mW1_1, mW1_2, mW2_1, mW2_2, aw2, fW, fb2)

    et, vals, w = pl.pallas_call(
        _topk_body,
        grid=(N // BLK_B,),
        in_specs=[
            pl.BlockSpec((HID, BLK_B), lambda i: (0, i)),
            whole((HID, N)),
            pl.BlockSpec((N, 1), lambda i: (0, 0)),
        ],
        out_specs=[
            pl.BlockSpec((16, BLK_B), lambda i: (0, i)),
            pl.BlockSpec((BLK_B, 1), lambda i: (i, 0)),
            pl.BlockSpec((BLK_B, 1), lambda i: (i, 0)),
        ],
        out_shape=[
            jax.ShapeDtypeStruct((16, N), jnp.int32),
            jax.ShapeDtypeStruct((N, 1), f32),
            jax.ShapeDtypeStruct((N, 1), f32),
        ],
    )(normed_t, normed_t, rsum)

    keepf = vals.reshape(1, N)

    Hmat = pl.pallas_call(
        _hbuild_body,
        grid=(N // BLK_C,),
        in_specs=[whole((16, N)), whole((1, N))],
        out_specs=pl.BlockSpec((BLK_C, N), lambda i: (i, 0)),
        out_shape=jax.ShapeDtypeStruct((N, N), f32),
    )(et, keepf)

    return Hmat, w.reshape(N)


# speedup vs baseline: 1.0710x; 1.0710x over previous
import jax
import jax.numpy as jnp
from jax import lax
from jax.experimental import pallas as pl

N = 4096
HID = 256
TOP_K = 10
BLK_A = 512
BLK_B = 512
BLK_C = 512
SENT = -1e9


def _dense_body(x0, x1, x2, w01, w02, w11, w12, w21, w22, aw, fw, fb,
                nt_out, rsum_out):
    a = aw[...]
    a = a - jnp.max(a, axis=1, keepdims=True)
    e = jnp.exp(a)
    a = e / jnp.sum(e, axis=1, keepdims=True)

    def mlp(x, w1, w2):
        h = jnp.maximum(jnp.dot(x[...], w1[...], preferred_element_type=jnp.float32), 0.0)
        return jnp.dot(h, w2[...], preferred_element_type=jnp.float32)

    f0 = mlp(x0, w01, w02)
    f1 = mlp(x1, w11, w12)
    f2 = mlp(x2, w21, w22)
    fsum = a[0, 0] * f0 + a[0, 1] * f1 + a[0, 2] * f2
    fused = jnp.dot(fsum, fw[...], preferred_element_type=jnp.float32) + fb[...]
    nrm = jnp.sqrt(jnp.sum(fused * fused, axis=1, keepdims=True))
    nrm = jnp.maximum(nrm, 1e-12)
    nt_out[...] = (fused / nrm).T
    rsum_out[...] = jnp.sum(fused, axis=1, keepdims=True)


def _topk_body(nb, nt, rsum, et_out, vals_out, w_out):
    i = pl.program_id(0)
    r0 = i * BLK_B
    rids_i = r0 + lax.broadcasted_iota(jnp.int32, (BLK_B, 1), 0)
    rids = rids_i.astype(jnp.float32)
    cols = lax.broadcasted_iota(jnp.int32, (BLK_B, N), 1).astype(jnp.float32)
    sim = lax.dot_general(nb[...], nt[...], (((0,), (0,)), ((), ())),
                          preferred_element_type=jnp.float32)
    sim = jnp.where(cols == rids, SENT, sim)
    picks = [rids]
    for _ in range(TOP_K):
        m = jnp.max(sim, axis=1, keepdims=True)
        cand = jnp.where(sim == m, cols, float(N))
        j = jnp.min(cand, axis=1, keepdims=True)
        sim = jnp.where(cols == j, SENT, sim)
        picks.append(j)
    selmask = (sim == SENT).astype(jnp.float32)
    msum = jnp.dot(selmask, rsum[...], preferred_element_type=jnp.float32)
    w = jax.nn.sigmoid(msum / float((TOP_K + 1) * HID))
    keep = w > 0.0
    picks += [jnp.full((BLK_B, 1), -1.0, jnp.float32)] * (16 - len(picks))
    edges = jnp.concatenate(picks, axis=1).astype(jnp.int32)
    et_out[...] = edges.T
    vals_out[...] = jnp.where(keep, 1.0, 0.0)
    w_out[...] = jnp.where(keep, w, 0.0)


def _hbuild_body(et, keepf, h_out):
    i = pl.program_id(0)
    r0 = i * BLK_C
    rids = (r0 + lax.broadcasted_iota(jnp.int32, (BLK_C, 1), 0)).astype(jnp.float32)
    kf = keepf[...]
    h = jnp.where(et[0:1, :].astype(jnp.float32) == rids, kf, 0.0)
    for j in range(1, TOP_K + 1):
        nj = et[j:j + 1, :].astype(jnp.float32)
        h = jnp.maximum(h, jnp.where(nj == rids, kf, 0.0))
    h_out[...] = h


def kernel(x0, x1, x2, mW0_1, mb0_1, mW0_2, mb0_2, mW1_1, mb1_1, mW1_2, mb1_2,
           mW2_1, mb2_1, mW2_2, mb2_2, attn_weights, fW, fb):
    f32 = jnp.float32
    aw2 = attn_weights.reshape(1, 3)
    fb2 = fb.reshape(1, HID)

    whole = lambda shape: pl.BlockSpec(shape, lambda i: (0, 0))
    rows = lambda w: pl.BlockSpec((BLK_A, w), lambda i: (i, 0))

    normed_t, rsum = pl.pallas_call(
        _dense_body,
        grid=(N // BLK_A,),
        in_specs=[
            rows(256), rows(512), rows(128),
            whole((256, HID)), whole((HID, HID)),
            whole((512, HID)), whole((HID, HID)),
            whole((128, HID)), whole((HID, HID)),
            whole((1, 3)), whole((HID, HID)), whole((1, HID)),
        ],
        out_specs=[pl.BlockSpec((HID, BLK_A), lambda i: (0, i)), rows(1)],
        out_shape=[
            jax.ShapeDtypeStruct((HID, N), f32),
            jax.ShapeDtypeStruct((N, 1), f32),
        ],
    )(x0, x1, x2, mW0_1, mW0_2, mW1_1, mW1_2, mW2_1, mW2_2, aw2, fW, fb2)

    et, vals, w = pl.pallas_call(
        _topk_body,
        grid=(N // BLK_B,),
        in_specs=[
            pl.BlockSpec((HID, BLK_B), lambda i: (0, i)),
            whole((HID, N)),
            pl.BlockSpec((N, 1), lambda i: (0, 0)),
        ],
        out_specs=[
            pl.BlockSpec((16, BLK_B), lambda i: (0, i)),
            pl.BlockSpec((BLK_B, 1), lambda i: (i, 0)),
            pl.BlockSpec((BLK_B, 1), lambda i: (i, 0)),
        ],
        out_shape=[
            jax.ShapeDtypeStruct((16, N), jnp.int32),
            jax.ShapeDtypeStruct((N, 1), f32),
            jax.ShapeDtypeStruct((N, 1), f32),
        ],
    )(normed_t, normed_t, rsum)

    keepf = vals.reshape(1, N)

    Hmat = pl.pallas_call(
        _hbuild_body,
        grid=(N // BLK_C,),
        in_specs=[whole((16, N)), whole((1, N))],
        out_specs=pl.BlockSpec((BLK_C, N), lambda i: (i, 0)),
        out_shape=jax.ShapeDtypeStruct((N, N), f32),
    )(et, keepf)

    return Hmat, w.reshape(N)


# speedup vs baseline: 1.1048x; 1.0316x over previous
import jax
import jax.numpy as jnp
from jax import lax
from jax.experimental import pallas as pl
from jax.experimental.pallas import tpu as pltpu

N = 4096
HID = 256
TOP_K = 10
BLK = 512
NBLK = N // BLK
SENT = -1e9


def _body(x0, x1, x2, w01, w02, w11, w12, w21, w22, aw, fw, fb,
          h_out, w_out, nrm_s, nt_s, rsum_s, et_s, keep_s):
    i = pl.program_id(0)

    @pl.when(i < NBLK)
    def _phase_a():
        a = aw[...]
        a = a - jnp.max(a, axis=1, keepdims=True)
        e = jnp.exp(a)
        a = e / jnp.sum(e, axis=1, keepdims=True)

        def mlp(x, w1, w2):
            h = jnp.maximum(jnp.dot(x[...], w1[...], preferred_element_type=jnp.float32), 0.0)
            return jnp.dot(h, w2[...], preferred_element_type=jnp.float32)

        fsum = (a[0, 0] * mlp(x0, w01, w02) + a[0, 1] * mlp(x1, w11, w12)
                + a[0, 2] * mlp(x2, w21, w22))
        fused = jnp.dot(fsum, fw[...], preferred_element_type=jnp.float32) + fb[...]
        nrm = jnp.sqrt(jnp.sum(fused * fused, axis=1, keepdims=True))
        nrm = jnp.maximum(nrm, 1e-12)
        normed = fused / nrm
        nrm_s[pl.ds(i * BLK, BLK), :] = normed
        nt_s[:, pl.ds(i, 1), :] = normed.T.reshape(HID, 1, BLK)
        rsum_s[pl.ds(i * BLK, BLK), :] = jnp.sum(fused, axis=1, keepdims=True)

    @pl.when(jnp.logical_and(i >= NBLK, i < 2 * NBLK))
    def _phase_b():
        j = i - NBLK
        r0 = j * BLK
        rids_i = r0 + lax.broadcasted_iota(jnp.int32, (BLK, 1), 0)
        rids = rids_i.astype(jnp.float32)
        cols = lax.broadcasted_iota(jnp.int32, (BLK, N), 1).astype(jnp.float32)
        nb = nrm_s[pl.ds(r0, BLK), :]
        nt = nt_s[...].reshape(HID, N)
        sim = jnp.dot(nb, nt, preferred_element_type=jnp.float32)
        sim = jnp.where(cols == rids, SENT, sim)
        picks = [rids]
        for _ in range(TOP_K):
            m = jnp.max(sim, axis=1, keepdims=True)
            cand = jnp.where(sim == m, cols, float(N))
            jj = jnp.min(cand, axis=1, keepdims=True)
            sim = jnp.where(cols == jj, SENT, sim)
            picks.append(jj)
        selmask = (sim == SENT).astype(jnp.float32)
        msum = jnp.dot(selmask, rsum_s[...], preferred_element_type=jnp.float32)
        w = jax.nn.sigmoid(msum / float((TOP_K + 1) * HID))
        keep = w > 0.0
        picks += [jnp.full((BLK, 1), -1.0, jnp.float32)] * (16 - len(picks))
        edges = jnp.concatenate(picks, axis=1).astype(jnp.int32)
        et_s[:, pl.ds(j, 1), :] = edges.T.reshape(16, 1, BLK)
        keep_s[pl.ds(j, 1), :] = jnp.where(keep, 1.0, 0.0).T.reshape(1, BLK)
        w_out[...] = jnp.where(keep, w, 0.0).T.reshape(1, BLK)

    @pl.when(i >= 2 * NBLK)
    def _phase_c():
        j = i - 2 * NBLK
        r0 = j * BLK
        rids = (r0 + lax.broadcasted_iota(jnp.int32, (BLK, 1), 0)).astype(jnp.float32)
        for g in range(NBLK):
            et_g = et_s[:, g, :].astype(jnp.float32)
            kf = keep_s[g:g + 1, :]
            h = jnp.where(et_g[0:1, :] == rids, kf, 0.0)
            for t in range(1, TOP_K + 1):
                h = jnp.maximum(h, jnp.where(et_g[t:t + 1, :] == rids, kf, 0.0))
            h_out[:, g * BLK:(g + 1) * BLK] = h


def kernel(x0, x1, x2, mW0_1, mb0_1, mW0_2, mb0_2, mW1_1, mb1_1, mW1_2, mb1_2,
           mW2_1, mb2_1, mW2_2, mb2_2, attn_weights, fW, fb):
    f32 = jnp.float32
    aw2 = attn_weights.reshape(1, 3)
    fb2 = fb.reshape(1, HID)

    whole = lambda shape: pl.BlockSpec(shape, lambda i: tuple(0 for _ in shape))
    rows = lambda w: pl.BlockSpec((BLK, w), lambda i: (jnp.minimum(i, NBLK - 1), 0))

    Hmat, w = pl.pallas_call(
        _body,
        grid=(3 * NBLK,),
        in_specs=[
            rows(256), rows(512), rows(128),
            whole((256, HID)), whole((HID, HID)),
            whole((512, HID)), whole((HID, HID)),
            whole((128, HID)), whole((HID, HID)),
            whole((1, 3)), whole((HID, HID)), whole((1, HID)),
        ],
        out_specs=[
            pl.BlockSpec((BLK, N), lambda i: (jnp.clip(i - 2 * NBLK, 0, NBLK - 1), 0)),
            pl.BlockSpec((1, BLK), lambda i: (0, jnp.clip(i - NBLK, 0, NBLK - 1))),
        ],
        out_shape=[
            jax.ShapeDtypeStruct((N, N), f32),
            jax.ShapeDtypeStruct((1, N), f32),
        ],
        scratch_shapes=[
            pltpu.VMEM((N, HID), f32),
            pltpu.VMEM((HID, NBLK, BLK), f32),
            pltpu.VMEM((N, 1), f32),
            pltpu.VMEM((16, NBLK, BLK), jnp.int32),
            pltpu.VMEM((NBLK, BLK), f32),
        ],
    )(x0, x1, x2, mW0_1, mW0_2, mW1_1, mW1_2, mW2_1, mW2_2, aw2, fW, fb2)

    return Hmat, w.reshape(N)
